# baseline (speedup 1.0000x reference)
import functools

import jax
import jax.numpy as jnp
from jax import lax
from jax.experimental import pallas as pl
from jax.experimental.pallas import tpu as pltpu
from jax.experimental.pallas import tpu_sc as plsc

N = 10000
E = 160000
D_IN = 512
D_OUT = 1433
NC = 12
DP = NC * 128
EPS = 1e-15
MAXNORM = 1.0 - 1e-3

ROWS_A = 400

EPT = E // 16
G = 80
NB = EPT // G
NBUF = 2


def _artanh(z):
    z = jnp.clip(z, -1.0 + 1e-7, 1.0 - 1e-7)
    return 0.5 * jnp.log((1.0 + z) / (1.0 - z))


def _rownorm(v):
    return jnp.maximum(jnp.sqrt(jnp.sum(v * v, axis=-1, keepdims=True)), EPS)


def _proj(v):
    n = _rownorm(v)
    return jnp.where(n > MAXNORM, v / n * MAXNORM, v)


def _h_body(x_ref, w_ref, b_ref, o_ref):
    x = x_ref[...]
    w = w_ref[...]
    b = b_ref[...]
    xn = _rownorm(x)
    xt = x / xn * _artanh(xn)
    xtn = _rownorm(xt)
    mx = jnp.dot(xt, w, preferred_element_type=jnp.float32)
    mxn = _rownorm(mx)
    res = jnp.tanh(mxn / xtn * _artanh(xtn)) * mx / mxn
    cond = jnp.sum(jnp.abs(mx), axis=-1, keepdims=True) < EPS
    h = jnp.where(cond, 0.0, res)
    h = _proj(h)
    bn = _rownorm(b)
    hb = _proj(jnp.tanh(bn) * b / bn)
    x2 = jnp.sum(h * h, axis=-1, keepdims=True)
    y2 = jnp.sum(hb * hb, axis=-1, keepdims=True)
    xy = jnp.sum(h * hb, axis=-1, keepdims=True)
    num = (1.0 + 2.0 * xy + y2) * h + (1.0 - x2) * hb
    den = 1.0 + 2.0 * xy + x2 * y2
    h = _proj(num / jnp.maximum(den, EPS))
    for j in range(NC):
        o_ref[j] = h[:, j * 128:(j + 1) * 128]


def _compute_h(x, w_pad, b_pad, interpret=False):
    return pl.pallas_call(
        _h_body,
        grid=(N // ROWS_A,),
        in_specs=[
            pl.BlockSpec((ROWS_A, D_IN), lambda i: (i, 0)),
            pl.BlockSpec((D_IN, DP), lambda i: (0, 0)),
            pl.BlockSpec((1, DP), lambda i: (0, 0)),
        ],
        out_specs=pl.BlockSpec((NC, ROWS_A, 128), lambda i: (0, i, 0)),
        out_shape=jax.ShapeDtypeStruct((NC, N, 128), jnp.float32),
        interpret=interpret,
    )(x, w_pad, b_pad)


def _post_body(s_ref, o_ref):
    u = jnp.concatenate([s_ref[j] for j in range(NC)], axis=1)
    un = _rownorm(u)
    o1 = _proj(jnp.tanh(un) * u / un)
    n2 = _rownorm(o1)
    t = jax.nn.relu(o1 / n2 * _artanh(n2))
    tn = _rownorm(t)
    o2 = _proj(jnp.tanh(tn) * t / tn)
    o_ref[...] = o2[:, :D_OUT]


def _post(supp, interpret=False):
    return pl.pallas_call(
        _post_body,
        grid=(N // ROWS_A,),
        in_specs=[pl.BlockSpec((NC, ROWS_A, 128), lambda i: (0, i, 0))],
        out_specs=pl.BlockSpec((ROWS_A, D_OUT), lambda i: (i, 0)),
        out_shape=jax.ShapeDtypeStruct((N, D_OUT), jnp.float32),
        interpret=interpret,
    )(supp)


def _sc_agg(h3, row, col):
    mesh = plsc.VectorSubcoreMesh(core_axis_name="c", subcore_axis_name="s")

    @functools.partial(
        pl.kernel,
        out_type=jax.ShapeDtypeStruct((NC, N, 128), jnp.float32),
        mesh=mesh,
        scratch_types=[
            pltpu.VMEM((EPT,), jnp.int32),
            pltpu.VMEM((NB, G), jnp.int32),
            pltpu.VMEM((NBUF * G, 128), jnp.float32),
            pltpu.VMEM((16, 128), jnp.float32),
            pltpu.VMEM_SHARED((N, 128), jnp.float32),
            pltpu.SemaphoreType.DMA,
            pltpu.SemaphoreType.DMA,
        ],
    )
    def agg(h_hbm, row_hbm, col_hbm, out_hbm,
            col_v, rowc, stage, zbuf, accum, gsem, ssem):
        cid = lax.axis_index("c")
        sid = lax.axis_index("s")
        base_e = sid * EPT
        pltpu.sync_copy(col_hbm.at[pl.ds(base_e, EPT)], col_v)
        pltpu.sync_copy(row_hbm.at[sid], rowc)

        zeros16 = jnp.zeros((16,), jnp.float32)

        def zb_row(r, carry):
            def zb_col(k, c2):
                zbuf[r, pl.ds(k * 16, 16)] = zeros16
                return c2
            return lax.fori_loop(0, 8, zb_col, carry)
        lax.fori_loop(0, 16, zb_row, 0)

        for p in range(NC // 2):
            cc = p * 2 + cid
            h_c = h_hbm.at[cc]
            out_c = out_hbm.at[cc]

            def clr_start(k, carry):
                pltpu.async_copy(zbuf,
                                 accum.at[pl.ds(sid * 624 + k * 16, 16)],
                                 ssem)
                return carry
            lax.fori_loop(0, 39, clr_start, 0)

            @pl.when(sid == 0)
            def _():
                pltpu.sync_copy(zbuf, accum.at[pl.ds(9984, 16)])

            def clr_fin(k, carry):
                pltpu.make_async_copy(
                    zbuf, accum.at[pl.ds(sid * 624 + k * 16, 16)],
                    ssem).wait()
                return carry
            lax.fori_loop(0, 39, clr_fin, 0)
            plsc.subcore_barrier()

            slot0 = stage.at[pl.ds(0, G)]
            slot1 = stage.at[pl.ds(G, G)]

            def start_g(b, stg, sem):
                @pl.when(b < NB)
                def _():
                    pltpu.async_copy(h_c.at[col_v.at[pl.ds(b * G, G)]],
                                     stg, sem)

            def fin_g(b, stg, sem):
                @pl.when(b < NB)
                def _():
                    pltpu.make_async_copy(h_c.at[pl.ds(0, G)],
                                          stg, sem).wait()
                    pltpu.sync_copy(stg, accum.at[rowc.at[b]], add=True)

            start_g(0, slot0, gsem)

            def pipe(j, carry):
                b0 = 2 * j
                start_g(b0 + 1, slot1, ssem)
                fin_g(b0, slot0, gsem)
                start_g(b0 + 2, slot0, gsem)
                fin_g(b0 + 1, slot1, ssem)
                return carry
            lax.fori_loop(0, (NB + 1) // 2, pipe, 0)
            plsc.subcore_barrier()

            pltpu.sync_copy(accum.at[pl.ds(sid * 624, 624)],
                            out_c.at[pl.ds(sid * 624, 624)])

            @pl.when(sid == 0)
            def _():
                pltpu.sync_copy(accum.at[pl.ds(9984, 16)],
                                out_c.at[pl.ds(9984, 16)])

    return agg(h3, row, col)


def kernel(x, edge_index, W, b):
    w_pad = jnp.pad(W, ((0, 0), (0, DP - D_OUT)))
    b_pad = jnp.pad(b, (0, DP - D_OUT)).reshape(1, DP)
    h3 = _compute_h(x, w_pad, b_pad)
    row2 = edge_index[0].reshape(16, NB, G)
    supp = _sc_agg(h3, row2, edge_index[1])
    return _post(supp)


# speedup vs baseline: 1.1432x; 1.1432x over previous
import functools

import jax
import jax.numpy as jnp
from jax import lax
from jax.experimental import pallas as pl
from jax.experimental.pallas import tpu as pltpu
from jax.experimental.pallas import tpu_sc as plsc

N = 10000
E = 160000
D_IN = 512
D_OUT = 1433
NC = 12
DP = NC * 128
EPS = 1e-15
MAXNORM = 1.0 - 1e-3

ROWS_A = 400

EPT = E // 16
G = 80
NB = EPT // G
NBUF = 3


def _artanh(z):
    z = jnp.clip(z, -1.0 + 1e-7, 1.0 - 1e-7)
    return 0.5 * jnp.log((1.0 + z) / (1.0 - z))


def _rownorm(v):
    return jnp.maximum(jnp.sqrt(jnp.sum(v * v, axis=-1, keepdims=True)), EPS)


def _proj(v):
    n = _rownorm(v)
    return jnp.where(n > MAXNORM, v / n * MAXNORM, v)


def _h_body(x_ref, w_ref, b_ref, o_ref):
    x = x_ref[...]
    w = w_ref[...]
    b = b_ref[...]
    xn = _rownorm(x)
    xt = x / xn * _artanh(xn)
    xtn = _rownorm(xt)
    mx = jnp.dot(xt, w, preferred_element_type=jnp.float32)
    mxn = _rownorm(mx)
    res = jnp.tanh(mxn / xtn * _artanh(xtn)) * mx / mxn
    cond = jnp.sum(jnp.abs(mx), axis=-1, keepdims=True) < EPS
    h = jnp.where(cond, 0.0, res)
    h = _proj(h)
    bn = _rownorm(b)
    hb = _proj(jnp.tanh(bn) * b / bn)
    x2 = jnp.sum(h * h, axis=-1, keepdims=True)
    y2 = jnp.sum(hb * hb, axis=-1, keepdims=True)
    xy = jnp.sum(h * hb, axis=-1, keepdims=True)
    num = (1.0 + 2.0 * xy + y2) * h + (1.0 - x2) * hb
    den = 1.0 + 2.0 * xy + x2 * y2
    h = _proj(num / jnp.maximum(den, EPS))
    for j in range(NC):
        o_ref[j] = h[:, j * 128:(j + 1) * 128]


def _compute_h(x, w_pad, b_pad, interpret=False):
    return pl.pallas_call(
        _h_body,
        grid=(N // ROWS_A,),
        in_specs=[
            pl.BlockSpec((ROWS_A, D_IN), lambda i: (i, 0)),
            pl.BlockSpec((D_IN, DP), lambda i: (0, 0)),
            pl.BlockSpec((1, DP), lambda i: (0, 0)),
        ],
        out_specs=pl.BlockSpec((NC, ROWS_A, 128), lambda i: (0, i, 0)),
        out_shape=jax.ShapeDtypeStruct((NC, N, 128), jnp.float32),
        interpret=interpret,
    )(x, w_pad, b_pad)


def _post_body(s_ref, o_ref):
    u = jnp.concatenate([s_ref[j] for j in range(NC)], axis=1)
    un = _rownorm(u)
    o1 = _proj(jnp.tanh(un) * u / un)
    n2 = _rownorm(o1)
    t = jax.nn.relu(o1 / n2 * _artanh(n2))
    tn = _rownorm(t)
    o2 = _proj(jnp.tanh(tn) * t / tn)
    o_ref[...] = o2[:, :D_OUT]


def _post(supp, interpret=False):
    return pl.pallas_call(
        _post_body,
        grid=(N // ROWS_A,),
        in_specs=[pl.BlockSpec((NC, ROWS_A, 128), lambda i: (0, i, 0))],
        out_specs=pl.BlockSpec((ROWS_A, D_OUT), lambda i: (i, 0)),
        out_shape=jax.ShapeDtypeStruct((N, D_OUT), jnp.float32),
        interpret=interpret,
    )(supp)


def _sc_agg(h3, pk):
    mesh = plsc.VectorSubcoreMesh(core_axis_name="c", subcore_axis_name="s")

    @functools.partial(
        pl.kernel,
        out_type=jax.ShapeDtypeStruct((NC, N, 128), jnp.float32),
        mesh=mesh,
        scratch_types=[
            pltpu.VMEM((EPT,), jnp.int32),
            pltpu.VMEM((NBUF, G), jnp.int32),
            pltpu.VMEM((NBUF, G), jnp.int32),
            pltpu.VMEM((NBUF * G, 128), jnp.float32),
            pltpu.VMEM((16, 128), jnp.float32),
            pltpu.VMEM_SHARED((N, 128), jnp.float32),
            [pltpu.SemaphoreType.DMA for _ in range(NBUF)],
            pltpu.SemaphoreType.DMA,
        ],
    )
    def agg(h_hbm, pk_hbm, out_hbm,
            pkv, idxc, idxr, stage, zbuf, accum, gsems, ssem):
        cid = lax.axis_index("c")
        sid = lax.axis_index("s")
        base_e = sid * EPT
        pltpu.sync_copy(pk_hbm.at[pl.ds(base_e, EPT)], pkv)

        zeros16 = jnp.zeros((16,), jnp.float32)

        def zb_row(r, carry):
            def zb_col(k, c2):
                zbuf[r, pl.ds(k * 16, 16)] = zeros16
                return c2
            return lax.fori_loop(0, 8, zb_col, carry)
        lax.fori_loop(0, 16, zb_row, 0)

        for p in range(NC // 2):
            cc = p * 2 + cid
            h_c = h_hbm.at[cc]
            out_c = out_hbm.at[cc]

            def clr_start(k, carry):
                pltpu.async_copy(zbuf,
                                 accum.at[pl.ds(sid * 624 + k * 16, 16)],
                                 ssem)
                return carry
            lax.fori_loop(0, 39, clr_start, 0)

            @pl.when(sid == 0)
            def _():
                pltpu.sync_copy(zbuf, accum.at[pl.ds(9984, 16)])

            def clr_fin(k, carry):
                pltpu.make_async_copy(
                    zbuf, accum.at[pl.ds(sid * 624 + k * 16, 16)],
                    ssem).wait()
                return carry
            lax.fori_loop(0, 39, clr_fin, 0)
            plsc.subcore_barrier()

            slots = [stage.at[pl.ds(k * G, G)] for k in range(NBUF)]

            def conv(b, k):
                for g in range(G // 16):
                    v = pkv[pl.ds(b * G + g * 16, 16)]
                    idxc[k, pl.ds(g * 16, 16)] = \
                        lax.shift_right_logical(v, 16)
                    idxr[k, pl.ds(g * 16, 16)] = v & 0xFFFF

            def start_g(b, k):
                @pl.when(b < NB)
                def _():
                    conv(b, k)
                    pltpu.async_copy(h_c.at[idxc.at[k]], slots[k], gsems[k])

            for k in range(NBUF - 1):
                start_g(k, k)

            def pipe(j, carry):
                for k in range(NBUF):
                    i = NBUF * j + k

                    @pl.when(i < NB)
                    def _():
                        pltpu.make_async_copy(h_c.at[pl.ds(0, G)],
                                              slots[k], gsems[k]).wait()
                        pltpu.async_copy(slots[k], accum.at[idxr.at[k]],
                                         ssem, add=True)

                        @pl.when(i >= 1)
                        def _():
                            pltpu.make_async_copy(
                                slots[(k + 2) % NBUF],
                                accum.at[pl.ds(0, G)], ssem).wait()
                        start_g(i + NBUF - 1, (k + 2) % NBUF)
                return carry
            lax.fori_loop(0, (NB + NBUF - 1) // NBUF, pipe, 0)
            pltpu.make_async_copy(slots[(NB - 1) % NBUF],
                                  accum.at[pl.ds(0, G)], ssem).wait()
            plsc.subcore_barrier()

            pltpu.sync_copy(accum.at[pl.ds(sid * 624, 624)],
                            out_c.at[pl.ds(sid * 624, 624)])

            @pl.when(sid == 0)
            def _():
                pltpu.sync_copy(accum.at[pl.ds(9984, 16)],
                                out_c.at[pl.ds(9984, 16)])

    return agg(h3, pk)


def kernel(x, edge_index, W, b):
    w_pad = jnp.pad(W, ((0, 0), (0, DP - D_OUT)))
    b_pad = jnp.pad(b, (0, DP - D_OUT)).reshape(1, DP)
    h3 = _compute_h(x, w_pad, b_pad)
    pk = (edge_index[1].astype(jnp.int32) << 16) | edge_index[0]
    supp = _sc_agg(h3, pk)
    return _post(supp)


# speedup vs baseline: 1.1521x; 1.0078x over previous
import functools

import jax
import jax.numpy as jnp
from jax import lax
from jax.experimental import pallas as pl
from jax.experimental.pallas import tpu as pltpu
from jax.experimental.pallas import tpu_sc as plsc

N = 10000
E = 160000
D_IN = 512
D_OUT = 1433
NC = 12
DP = NC * 128
EPS = 1e-15
MAXNORM = 1.0 - 1e-3

ROWS_A = 1000

EPT = E // 16
G = 80
NB = EPT // G
NBUF = 3


def _artanh(z):
    z = jnp.clip(z, -1.0 + 1e-7, 1.0 - 1e-7)
    return 0.5 * jnp.log((1.0 + z) / (1.0 - z))


def _rownorm(v):
    return jnp.maximum(jnp.sqrt(jnp.sum(v * v, axis=-1, keepdims=True)), EPS)


def _proj(v):
    n = _rownorm(v)
    return jnp.where(n > MAXNORM, v / n * MAXNORM, v)


def _h_body(x_ref, w_ref, b_ref, o_ref):
    x = x_ref[...]
    w = w_ref[...]
    b = b_ref[...]
    xn = _rownorm(x)
    xt = x / xn * _artanh(xn)
    xtn = _rownorm(xt)
    mx = jnp.dot(xt, w, preferred_element_type=jnp.float32)
    mxn = _rownorm(mx)
    res = jnp.tanh(mxn / xtn * _artanh(xtn)) * mx / mxn
    cond = jnp.sum(jnp.abs(mx), axis=-1, keepdims=True) < EPS
    h = jnp.where(cond, 0.0, res)
    h = _proj(h)
    bn = _rownorm(b)
    hb = _proj(jnp.tanh(bn) * b / bn)
    x2 = jnp.sum(h * h, axis=-1, keepdims=True)
    y2 = jnp.sum(hb * hb, axis=-1, keepdims=True)
    xy = jnp.sum(h * hb, axis=-1, keepdims=True)
    num = (1.0 + 2.0 * xy + y2) * h + (1.0 - x2) * hb
    den = 1.0 + 2.0 * xy + x2 * y2
    h = _proj(num / jnp.maximum(den, EPS))
    for j in range(NC):
        o_ref[j] = h[:, j * 128:(j + 1) * 128]


def _compute_h(x, w_pad, b_pad, interpret=False):
    return pl.pallas_call(
        _h_body,
        grid=(N // ROWS_A,),
        in_specs=[
            pl.BlockSpec((ROWS_A, D_IN), lambda i: (i, 0)),
            pl.BlockSpec((D_IN, DP), lambda i: (0, 0)),
            pl.BlockSpec((1, DP), lambda i: (0, 0)),
        ],
        out_specs=pl.BlockSpec((NC, ROWS_A, 128), lambda i: (0, i, 0)),
        out_shape=jax.ShapeDtypeStruct((NC, N, 128), jnp.float32),
        interpret=interpret,
    )(x, w_pad, b_pad)


def _post_body(s_ref, o_ref):
    u = jnp.concatenate([s_ref[j] for j in range(NC)], axis=1)
    un = _rownorm(u)
    o1 = _proj(jnp.tanh(un) * u / un)
    n2 = _rownorm(o1)
    t = jax.nn.relu(o1 / n2 * _artanh(n2))
    tn = _rownorm(t)
    o2 = _proj(jnp.tanh(tn) * t / tn)
    o_ref[...] = o2[:, :D_OUT]


def _post(supp, interpret=False):
    return pl.pallas_call(
        _post_body,
        grid=(N // ROWS_A,),
        in_specs=[pl.BlockSpec((NC, ROWS_A, 128), lambda i: (0, i, 0))],
        out_specs=pl.BlockSpec((ROWS_A, D_OUT), lambda i: (i, 0)),
        out_shape=jax.ShapeDtypeStruct((N, D_OUT), jnp.float32),
        interpret=interpret,
    )(supp)


def _sc_agg(h3, pk):
    mesh = plsc.VectorSubcoreMesh(core_axis_name="c", subcore_axis_name="s")

    @functools.partial(
        pl.kernel,
        out_type=jax.ShapeDtypeStruct((NC, N, 128), jnp.float32),
        mesh=mesh,
        scratch_types=[
            pltpu.VMEM((EPT,), jnp.int32),
            pltpu.VMEM((NBUF, G), jnp.int32),
            pltpu.VMEM((NBUF, G), jnp.int32),
            pltpu.VMEM((NBUF * G, 128), jnp.float32),
            pltpu.VMEM((16, 128), jnp.float32),
            pltpu.VMEM_SHARED((N, 128), jnp.float32),
            [pltpu.SemaphoreType.DMA for _ in range(NBUF)],
            pltpu.SemaphoreType.DMA,
        ],
    )
    def agg(h_hbm, pk_hbm, out_hbm,
            pkv, idxc, idxr, stage, zbuf, accum, gsems, ssem):
        cid = lax.axis_index("c")
        sid = lax.axis_index("s")
        base_e = sid * EPT
        pltpu.sync_copy(pk_hbm.at[pl.ds(base_e, EPT)], pkv)

        zeros16 = jnp.zeros((16,), jnp.float32)

        def zb_row(r, carry):
            def zb_col(k, c2):
                zbuf[r, pl.ds(k * 16, 16)] = zeros16
                return c2
            return lax.fori_loop(0, 8, zb_col, carry)
        lax.fori_loop(0, 16, zb_row, 0)

        for p in range(NC // 2):
            cc = p * 2 + cid
            h_c = h_hbm.at[cc]
            out_c = out_hbm.at[cc]

            def clr_start(k, carry):
                pltpu.async_copy(zbuf,
                                 accum.at[pl.ds(sid * 624 + k * 16, 16)],
                                 ssem)
                return carry
            lax.fori_loop(0, 39, clr_start, 0)

            @pl.when(sid == 0)
            def _():
                pltpu.sync_copy(zbuf, accum.at[pl.ds(9984, 16)])

            def clr_fin(k, carry):
                pltpu.make_async_copy(
                    zbuf, accum.at[pl.ds(sid * 624 + k * 16, 16)],
                    ssem).wait()
                return carry
            lax.fori_loop(0, 39, clr_fin, 0)
            plsc.subcore_barrier()

            slots = [stage.at[pl.ds(k * G, G)] for k in range(NBUF)]

            def conv(b, k):
                for g in range(G // 16):
                    v = pkv[pl.ds(b * G + g * 16, 16)]
                    idxc[k, pl.ds(g * 16, 16)] = \
                        lax.shift_right_logical(v, 16)
                    idxr[k, pl.ds(g * 16, 16)] = v & 0xFFFF

            def start_g(b, k):
                @pl.when(b < NB)
                def _():
                    conv(b, k)
                    pltpu.async_copy(h_c.at[idxc.at[k]], slots[k], gsems[k])

            for k in range(NBUF - 1):
                start_g(k, k)

            def pipe(j, carry):
                for k in range(NBUF):
                    i = NBUF * j + k

                    @pl.when(i < NB)
                    def _():
                        pltpu.make_async_copy(h_c.at[pl.ds(0, G)],
                                              slots[k], gsems[k]).wait()
                        pltpu.async_copy(slots[k], accum.at[idxr.at[k]],
                                         ssem, add=True)

                        @pl.when(i >= 1)
                        def _():
                            pltpu.make_async_copy(
                                slots[(k + 2) % NBUF],
                                accum.at[pl.ds(0, G)], ssem).wait()
                        start_g(i + NBUF - 1, (k + 2) % NBUF)
                return carry
            lax.fori_loop(0, (NB + NBUF - 1) // NBUF, pipe, 0)
            pltpu.make_async_copy(slots[(NB - 1) % NBUF],
                                  accum.at[pl.ds(0, G)], ssem).wait()
            plsc.subcore_barrier()

            pltpu.sync_copy(accum.at[pl.ds(sid * 624, 624)],
                            out_c.at[pl.ds(sid * 624, 624)])

            @pl.when(sid == 0)
            def _():
                pltpu.sync_copy(accum.at[pl.ds(9984, 16)],
                                out_c.at[pl.ds(9984, 16)])

    return agg(h3, pk)


def kernel(x, edge_index, W, b):
    w_pad = jnp.pad(W, ((0, 0), (0, DP - D_OUT)))
    b_pad = jnp.pad(b, (0, DP - D_OUT)).reshape(1, DP)
    h3 = _compute_h(x, w_pad, b_pad)
    pk = (edge_index[1].astype(jnp.int32) << 16) | edge_index[0]
    supp = _sc_agg(h3, pk)
    return _post(supp)


# speedup vs baseline: 1.2024x; 1.0437x over previous
import functools

import jax
import jax.numpy as jnp
from jax import lax
from jax.experimental import pallas as pl
from jax.experimental.pallas import tpu as pltpu
from jax.experimental.pallas import tpu_sc as plsc

N = 10000
E = 160000
D_IN = 512
D_OUT = 1433
NC = 12
DP = NC * 128
EPS = 1e-15
MAXNORM = 1.0 - 1e-3

ROWS_A = 1000

EPT = E // 16
G = 80
NB = EPT // G
NBUF = 3


def _artanh(z):
    z = jnp.clip(z, -1.0 + 1e-7, 1.0 - 1e-7)
    return 0.5 * jnp.log((1.0 + z) / (1.0 - z))


def _rownorm(v):
    return jnp.maximum(jnp.sqrt(jnp.sum(v * v, axis=-1, keepdims=True)), EPS)


def _proj(v):
    n = _rownorm(v)
    return jnp.where(n > MAXNORM, v / n * MAXNORM, v)


def _h_body(x_ref, w_ref, b_ref, o_ref):
    x = x_ref[...]
    w = w_ref[...]
    b = b_ref[...]
    xn = _rownorm(x)
    axn = _artanh(xn)
    xt = x / xn * axn
    xtn = jnp.maximum(axn, EPS)
    mx = jnp.dot(xt, w, preferred_element_type=jnp.float32)
    mxn = _rownorm(mx)
    g = jnp.tanh(mxn / xtn * _artanh(xtn))
    cond = jnp.sum(jnp.abs(mx), axis=-1, keepdims=True) < EPS
    gn = jnp.where(cond, 0.0, g)
    hs = jnp.where(gn > MAXNORM, MAXNORM / jnp.maximum(gn, EPS), 1.0)
    h = jnp.where(cond, 0.0, mx / mxn) * (g * hs)
    hn = gn * hs
    bn = _rownorm(b)
    hb = _proj(jnp.tanh(bn) * b / bn)
    x2 = hn * hn
    y2 = jnp.sum(hb * hb, axis=-1, keepdims=True)
    xy = jnp.sum(h * hb, axis=-1, keepdims=True)
    num = (1.0 + 2.0 * xy + y2) * h + (1.0 - x2) * hb
    den = 1.0 + 2.0 * xy + x2 * y2
    h = _proj(num / jnp.maximum(den, EPS))
    for j in range(NC):
        o_ref[j] = h[:, j * 128:(j + 1) * 128]


def _compute_h(x, w_pad, b_pad, interpret=False):
    return pl.pallas_call(
        _h_body,
        grid=(N // ROWS_A,),
        in_specs=[
            pl.BlockSpec((ROWS_A, D_IN), lambda i: (i, 0)),
            pl.BlockSpec((D_IN, DP), lambda i: (0, 0)),
            pl.BlockSpec((1, DP), lambda i: (0, 0)),
        ],
        out_specs=pl.BlockSpec((NC, ROWS_A, 128), lambda i: (0, i, 0)),
        out_shape=jax.ShapeDtypeStruct((NC, N, 128), jnp.float32),
        interpret=interpret,
    )(x, w_pad, b_pad)


def _post_body(s_ref, o_ref):
    u = jnp.concatenate([s_ref[j] for j in range(NC)], axis=1)
    un = _rownorm(u)
    s2 = jnp.maximum(jnp.minimum(jnp.tanh(un), MAXNORM), EPS)
    t = jax.nn.relu(u) * (_artanh(s2) / un)
    tn = _rownorm(t)
    f = jnp.tanh(tn)
    fs = jnp.where(f > MAXNORM, MAXNORM / jnp.maximum(f, EPS), 1.0)
    o2 = t / tn * (f * fs)
    o_ref[...] = o2[:, :D_OUT]


def _post(supp, interpret=False):
    return pl.pallas_call(
        _post_body,
        grid=(N // ROWS_A,),
        in_specs=[pl.BlockSpec((NC, ROWS_A, 128), lambda i: (0, i, 0))],
        out_specs=pl.BlockSpec((ROWS_A, D_OUT), lambda i: (i, 0)),
        out_shape=jax.ShapeDtypeStruct((N, D_OUT), jnp.float32),
        interpret=interpret,
    )(supp)


def _sc_agg(h3, pk):
    mesh = plsc.VectorSubcoreMesh(core_axis_name="c", subcore_axis_name="s")

    @functools.partial(
        pl.kernel,
        out_type=jax.ShapeDtypeStruct((NC, N, 128), jnp.float32),
        mesh=mesh,
        scratch_types=[
            pltpu.VMEM((EPT,), jnp.int32),
            pltpu.VMEM((NBUF, G), jnp.int32),
            pltpu.VMEM((NBUF, G), jnp.int32),
            pltpu.VMEM((NBUF * G, 128), jnp.float32),
            pltpu.VMEM((16, 128), jnp.float32),
            pltpu.VMEM_SHARED((N, 128), jnp.float32),
            [pltpu.SemaphoreType.DMA for _ in range(NBUF)],
            pltpu.SemaphoreType.DMA,
        ],
    )
    def agg(h_hbm, pk_hbm, out_hbm,
            pkv, idxc, idxr, stage, zbuf, accum, gsems, ssem):
        cid = lax.axis_index("c")
        sid = lax.axis_index("s")
        base_e = sid * EPT
        pltpu.sync_copy(pk_hbm.at[pl.ds(base_e, EPT)], pkv)

        zeros16 = jnp.zeros((16,), jnp.float32)

        def zb_row(r, carry):
            def zb_col(k, c2):
                zbuf[r, pl.ds(k * 16, 16)] = zeros16
                return c2
            return lax.fori_loop(0, 8, zb_col, carry)
        lax.fori_loop(0, 16, zb_row, 0)

        for p in range(NC // 2):
            cc = p * 2 + cid
            h_c = h_hbm.at[cc]
            out_c = out_hbm.at[cc]

            def clr_start(k, carry):
                pltpu.async_copy(zbuf,
                                 accum.at[pl.ds(sid * 624 + k * 16, 16)],
                                 ssem)
                return carry
            lax.fori_loop(0, 39, clr_start, 0)

            @pl.when(sid == 0)
            def _():
                pltpu.sync_copy(zbuf, accum.at[pl.ds(9984, 16)])

            def clr_fin(k, carry):
                pltpu.make_async_copy(
                    zbuf, accum.at[pl.ds(sid * 624 + k * 16, 16)],
                    ssem).wait()
                return carry
            lax.fori_loop(0, 39, clr_fin, 0)
            plsc.subcore_barrier()

            slots = [stage.at[pl.ds(k * G, G)] for k in range(NBUF)]

            def conv(b, k):
                for g in range(G // 16):
                    v = pkv[pl.ds(b * G + g * 16, 16)]
                    idxc[k, pl.ds(g * 16, 16)] = \
                        lax.shift_right_logical(v, 16)
                    idxr[k, pl.ds(g * 16, 16)] = v & 0xFFFF

            def start_g(b, k):
                @pl.when(b < NB)
                def _():
                    conv(b, k)
                    pltpu.async_copy(h_c.at[idxc.at[k]], slots[k], gsems[k])

            for k in range(NBUF - 1):
                start_g(k, k)

            def pipe(j, carry):
                for k in range(NBUF):
                    i = NBUF * j + k

                    @pl.when(i < NB)
                    def _():
                        pltpu.make_async_copy(h_c.at[pl.ds(0, G)],
                                              slots[k], gsems[k]).wait()
                        pltpu.async_copy(slots[k], accum.at[idxr.at[k]],
                                         ssem, add=True)

                        @pl.when(i >= 1)
                        def _():
                            pltpu.make_async_copy(
                                slots[(k + 2) % NBUF],
                                accum.at[pl.ds(0, G)], ssem).wait()
                        start_g(i + NBUF - 1, (k + 2) % NBUF)
                return carry
            lax.fori_loop(0, (NB + NBUF - 1) // NBUF, pipe, 0)
            pltpu.make_async_copy(slots[(NB - 1) % NBUF],
                                  accum.at[pl.ds(0, G)], ssem).wait()
            plsc.subcore_barrier()

            pltpu.sync_copy(accum.at[pl.ds(sid * 624, 624)],
                            out_c.at[pl.ds(sid * 624, 624)])

            @pl.when(sid == 0)
            def _():
                pltpu.sync_copy(accum.at[pl.ds(9984, 16)],
                                out_c.at[pl.ds(9984, 16)])

    return agg(h3, pk)


def kernel(x, edge_index, W, b):
    w_pad = jnp.pad(W, ((0, 0), (0, DP - D_OUT)))
    b_pad = jnp.pad(b, (0, DP - D_OUT)).reshape(1, DP)
    h3 = _compute_h(x, w_pad, b_pad)
    pk = (edge_index[1].astype(jnp.int32) << 16) | edge_index[0]
    supp = _sc_agg(h3, pk)
    return _post(supp)
